# baseline (speedup 1.0000x reference)
import functools

import jax
import jax.numpy as jnp
from jax import lax
from jax.experimental import pallas as pl
from jax.experimental.pallas import tpu as pltpu
from jax.experimental.pallas import tpu_sc as plsc

_L = 16


def _make_kwinners(B, C, H, W, CH, CHO):
  assert C % (2 * CH) == 0 and C % (2 * CHO) == 0 and W % _L == 0
  n_chunks = C // CH
  n_ochunks = C // CHO
  n_wgrp = W // _L
  HW = H * W
  mesh = plsc.VectorSubcoreMesh(core_axis_name="c", subcore_axis_name="s")

  @functools.partial(
      pl.kernel,
      mesh=mesh,
      out_type=jax.ShapeDtypeStruct((B, C, H, W), jnp.float32),
      compiler_params=pltpu.CompilerParams(use_tc_tiling_on_sc=True),
      scratch_types=[
          pltpu.VMEM((CH, H, W), jnp.float32),
          pltpu.VMEM((CH, H, W), jnp.float32),
          pltpu.VMEM((CHO, H, W), jnp.float32),
          pltpu.VMEM((CHO, H, W), jnp.float32),
          pltpu.VMEM((HW,), jnp.float32),
          pltpu.VMEM((HW,), jnp.int32),
          pltpu.SemaphoreType.DMA,
          pltpu.SemaphoreType.DMA,
          pltpu.SemaphoreType.DMA,
          pltpu.SemaphoreType.DMA,
      ],
  )
  def kw(x_hbm, o_hbm, ibuf_a, ibuf_b, obuf_a, obuf_b, rmax, ridx,
         rsem_a, rsem_b, wsem_a, wsem_b):
    b = lax.axis_index("s") * 2 + lax.axis_index("c")
    neg_inf = jnp.full((_L,), -jnp.inf, jnp.float32)
    zero_i = jnp.zeros((_L,), jnp.int32)
    zero_f = jnp.zeros((_L,), jnp.float32)
    ibufs = (ibuf_a, ibuf_b)
    rsems = (rsem_a, rsem_b)
    obufs = (obuf_a, obuf_b)
    wsems = (wsem_a, wsem_b)

    def init_g(g, _):
      col = pl.ds(g * _L, _L)
      rmax[col] = neg_inf
      ridx[col] = zero_i
      return 0

    lax.fori_loop(0, HW // _L, init_g, 0)

    pltpu.async_copy(x_hbm.at[b, pl.ds(0, CH), :, :], ibuf_a, rsem_a)
    pltpu.async_copy(x_hbm.at[b, pl.ds(CH, CH), :, :], ibuf_b, rsem_b)

    def read_body(g, _):
      for p in range(2):
        ci = 2 * g + p
        c0 = ci * CH
        pltpu.make_async_copy(
            x_hbm.at[b, pl.ds(c0, CH), :, :], ibufs[p], rsems[p]).wait()
        buf = ibufs[p]

        def red_h(h, _):
          for wj in range(n_wgrp):
            col = pl.ds(h * W + wj * _L, _L)
            m = rmax[col]
            i = ridx[col]
            for r in range(CH):
              v = buf[r, h, pl.ds(wj * _L, _L)]
              gt = v > m
              m = jnp.where(gt, v, m)
              i = jnp.where(gt, c0 + r, i)
            rmax[col] = m
            ridx[col] = i
          return 0

        lax.fori_loop(0, H, red_h, 0)

        @pl.when(ci + 2 < n_chunks)
        def _():
          pltpu.async_copy(
              x_hbm.at[b, pl.ds((ci + 2) * CH, CH), :, :], ibufs[p], rsems[p])

      return 0

    lax.fori_loop(0, n_chunks // 2, read_body, 0)

    def write_body(g, _):
      for p in range(2):
        ci = 2 * g + p
        c0 = ci * CHO

        @pl.when(g > 0)
        def _():
          pltpu.make_async_copy(
              obufs[p], o_hbm.at[b, pl.ds(c0, CHO), :, :], wsems[p]).wait()

        buf = obufs[p]

        def out_h(h, _):
          for wj in range(n_wgrp):
            col = pl.ds(h * W + wj * _L, _L)
            m = rmax[col]
            i = ridx[col]
            for r in range(CHO):
              buf[r, h, pl.ds(wj * _L, _L)] = jnp.where(i == c0 + r, m, zero_f)
          return 0

        lax.fori_loop(0, H, out_h, 0)
        pltpu.async_copy(buf, o_hbm.at[b, pl.ds(c0, CHO), :, :], wsems[p])

      return 0

    lax.fori_loop(0, n_ochunks // 2, write_body, 0)

    for p in range(2):
      last = n_ochunks - 2 + p
      pltpu.make_async_copy(
          obufs[p], o_hbm.at[b, pl.ds(last * CHO, CHO), :, :], wsems[p]).wait()

  return kw


def kernel(x, k):
  B, C, H, W = x.shape
  return _make_kwinners(B, C, H, W, 8, 6)(x)


# speedup vs baseline: 3.2357x; 3.2357x over previous
import functools

import jax
import jax.numpy as jnp
from jax import lax
from jax.experimental import pallas as pl
from jax.experimental.pallas import tpu as pltpu
from jax.experimental.pallas import tpu_sc as plsc

_L = 16


def _make_kwinners(B, H, W, C):
  assert C % _L == 0 and H % 2 == 0
  n_k = C // _L
  mesh = plsc.VectorSubcoreMesh(core_axis_name="c", subcore_axis_name="s")

  @functools.partial(
      pl.kernel,
      mesh=mesh,
      out_type=jax.ShapeDtypeStruct((B, H, W, C), jnp.float32),
      compiler_params=pltpu.CompilerParams(
          needs_layout_passes=False, use_tc_tiling_on_sc=True),
      scratch_types=[
          pltpu.VMEM((1, W, C), jnp.float32),
          pltpu.VMEM((1, W, C), jnp.float32),
          pltpu.VMEM((1, W, C), jnp.float32),
          pltpu.VMEM((1, W, C), jnp.float32),
          pltpu.SemaphoreType.DMA,
          pltpu.SemaphoreType.DMA,
          pltpu.SemaphoreType.DMA,
          pltpu.SemaphoreType.DMA,
      ],
  )
  def kw(x_hbm, o_hbm, ibuf_a, ibuf_b, obuf_a, obuf_b,
         rsem_a, rsem_b, wsem_a, wsem_b):
    b = lax.axis_index("s") * 2 + lax.axis_index("c")
    lane = lax.iota(jnp.int32, _L)
    big_i = jnp.full((_L,), C, jnp.int32)
    ibufs = (ibuf_a, ibuf_b)
    obufs = (obuf_a, obuf_b)
    rsems = (rsem_a, rsem_b)
    wsems = (wsem_a, wsem_b)

    pltpu.async_copy(x_hbm.at[b, pl.ds(0, 1), :, :], ibuf_a, rsem_a)
    pltpu.async_copy(x_hbm.at[b, pl.ds(1, 1), :, :], ibuf_b, rsem_b)

    def row_body(g, _):
      for p in range(2):
        h = 2 * g + p
        pltpu.make_async_copy(
            x_hbm.at[b, pl.ds(h, 1), :, :], ibufs[p], rsems[p]).wait()

        @pl.when(g > 0)
        def _():
          pltpu.make_async_copy(
              obufs[p], o_hbm.at[b, pl.ds(h - 2, 1), :, :], wsems[p]).wait()

        ib = ibufs[p]
        ob = obufs[p]

        def pix_body(w, _):
          m = ib[0, w, pl.ds(0, _L)]
          i = lane
          for kk in range(1, n_k):
            v = ib[0, w, pl.ds(kk * _L, _L)]
            gt = v > m
            m = jnp.where(gt, v, m)
            i = jnp.where(gt, kk * _L + lane, i)
          mx = jnp.max(m)
          wi = jnp.min(jnp.where(m == mx, i, big_i))
          for kk in range(n_k):
            cvec = kk * _L + lane
            ob[0, w, pl.ds(kk * _L, _L)] = jnp.where(
                cvec == wi, mx, jnp.float32(0))
          return 0

        lax.fori_loop(0, W, pix_body, 0)
        pltpu.async_copy(ob, o_hbm.at[b, pl.ds(h, 1), :, :], wsems[p])

        @pl.when(h + 2 < H)
        def _():
          pltpu.async_copy(
              x_hbm.at[b, pl.ds(h + 2, 1), :, :], ibufs[p], rsems[p])

      return 0

    lax.fori_loop(0, H // 2, row_body, 0)

    for p in range(2):
      pltpu.make_async_copy(
          obufs[p], o_hbm.at[b, pl.ds(H - 2 + p, 1), :, :], wsems[p]).wait()

  return kw


def kernel(x, k):
  B, C, H, W = x.shape
  xt = jnp.transpose(x, (0, 2, 3, 1))
  out_t = _make_kwinners(B, H, W, C)(xt)
  return jnp.transpose(out_t, (0, 3, 1, 2))
